# Initial kernel scaffold; baseline (speedup 1.0000x reference)
import jax
import jax.numpy as jnp
from jax.experimental import pallas as pl


def kernel(inputs, W):
    raise NotImplementedError("write your pallas kernel here")


# speedup vs baseline: 1.1022x; 1.1022x over previous
import functools

import jax
import jax.numpy as jnp
from jax import lax
from jax.experimental import pallas as pl
from jax.experimental.pallas import tpu as pltpu
from jax.experimental.pallas import tpu_sc as plsc

NUM_ROWS = 1_000_000
DIM = 32
BATCH = 16384 * 50

_NC = 2
_NS = 16
_NW = _NC * _NS
_PER_W = BATCH // _NW
_CHUNK = 1600
_N_CHUNKS = _PER_W // _CHUNK

_mesh = plsc.VectorSubcoreMesh(core_axis_name="c", subcore_axis_name="s")


@functools.partial(
    pl.kernel,
    mesh=_mesh,
    out_type=jax.ShapeDtypeStruct((BATCH, DIM), jnp.float32),
    scratch_types=[
        pltpu.VMEM((_CHUNK,), jnp.int32),
        pltpu.VMEM((_CHUNK, DIM), jnp.float32),
        pltpu.SemaphoreType.DMA,
    ],
    compiler_params=pltpu.CompilerParams(use_tc_tiling_on_sc=False),
)
def _gather_kernel(idx_hbm, table_hbm, out_hbm, idx_v, rows_v, sem):
    wid = lax.axis_index("s") * _NC + lax.axis_index("c")
    base = wid * _PER_W

    def body(i, carry):
        off = base + i * _CHUNK
        pltpu.sync_copy(idx_hbm.at[pl.ds(off, _CHUNK)], idx_v)
        pltpu.async_copy(table_hbm.at[idx_v], rows_v, sem).wait()
        pltpu.sync_copy(rows_v, out_hbm.at[pl.ds(off, _CHUNK)])
        return carry

    lax.fori_loop(0, _N_CHUNKS, body, 0)


def kernel(inputs, W):
    idx = inputs.reshape(-1).astype(jnp.int32)
    out = _gather_kernel(idx, W)
    return out.reshape(inputs.shape + (DIM,))


# speedup vs baseline: 1.1093x; 1.0064x over previous
import functools

import jax
import jax.numpy as jnp
from jax import lax
from jax.experimental import pallas as pl
from jax.experimental.pallas import tpu as pltpu
from jax.experimental.pallas import tpu_sc as plsc

NUM_ROWS = 1_000_000
DIM = 32
BATCH = 16384 * 50

_NC = 2
_NS = 16
_NW = _NC * _NS
_PER_W = BATCH // _NW
_CHUNK = 1600
_N_CHUNKS = _PER_W // _CHUNK
_NBUF = 2

_mesh = plsc.VectorSubcoreMesh(core_axis_name="c", subcore_axis_name="s")


@functools.partial(
    pl.kernel,
    mesh=_mesh,
    out_type=jax.ShapeDtypeStruct((BATCH, DIM), jnp.float32),
    scratch_types=[
        pltpu.VMEM((_NBUF, _CHUNK), jnp.int32),
        pltpu.VMEM((_NBUF, _CHUNK, DIM), jnp.float32),
        [pltpu.SemaphoreType.DMA] * _NBUF,
        [pltpu.SemaphoreType.DMA] * _NBUF,
        [pltpu.SemaphoreType.DMA] * _NBUF,
    ],
    compiler_params=pltpu.CompilerParams(use_tc_tiling_on_sc=False),
)
def _gather_kernel(idx_hbm, table_hbm, out_hbm, idx_v, rows_v, idx_sems,
                   gat_sems, out_sems):
    wid = lax.axis_index("s") * _NC + lax.axis_index("c")
    base = wid * _PER_W

    idx_d = [None] * _NBUF
    gat_d = [None] * _NBUF
    out_d = [None] * _NBUF

    def start_idx(i):
        b = i % _NBUF
        idx_d[b] = pltpu.async_copy(
            idx_hbm.at[pl.ds(base + i * _CHUNK, _CHUNK)], idx_v.at[b],
            idx_sems[b])

    def start_gather(i):
        b = i % _NBUF
        gat_d[b] = pltpu.async_copy(
            table_hbm.at[idx_v.at[b]], rows_v.at[b], gat_sems[b])

    def start_out(i):
        b = i % _NBUF
        out_d[b] = pltpu.async_copy(
            rows_v.at[b], out_hbm.at[pl.ds(base + i * _CHUNK, _CHUNK)],
            out_sems[b])

    for i in range(min(_NBUF, _N_CHUNKS)):
        start_idx(i)

    for i in range(_N_CHUNKS):
        b = i % _NBUF
        if out_d[b] is not None:
            out_d[b].wait()
        idx_d[b].wait()
        start_gather(i)
        gat_d[b].wait()
        start_out(i)
        if i + _NBUF < _N_CHUNKS:
            start_idx(i + _NBUF)

    for b in range(_NBUF):
        if out_d[b] is not None:
            out_d[b].wait()


def kernel(inputs, W):
    idx = inputs.reshape(-1).astype(jnp.int32)
    out = _gather_kernel(idx, W)
    return out.reshape(inputs.shape + (DIM,))


# speedup vs baseline: 1.1131x; 1.0035x over previous
import functools

import jax
import jax.numpy as jnp
from jax import lax
from jax.experimental import pallas as pl
from jax.experimental.pallas import tpu as pltpu
from jax.experimental.pallas import tpu_sc as plsc

NUM_ROWS = 1_000_000
DIM = 32
BATCH = 16384 * 50

_NC = 2
_NS = 16
_NW = _NC * _NS
_PER_W = BATCH // _NW
_CHUNK = 800
_N_CHUNKS = _PER_W // _CHUNK
_NBUF = 4

_mesh = plsc.VectorSubcoreMesh(core_axis_name="c", subcore_axis_name="s")


@functools.partial(
    pl.kernel,
    mesh=_mesh,
    out_type=jax.ShapeDtypeStruct((BATCH, DIM), jnp.float32),
    scratch_types=[
        pltpu.VMEM((_NBUF, _CHUNK), jnp.int32),
        pltpu.VMEM((_NBUF, _CHUNK, DIM), jnp.float32),
        [pltpu.SemaphoreType.DMA] * _NBUF,
        [pltpu.SemaphoreType.DMA] * _NBUF,
        [pltpu.SemaphoreType.DMA] * _NBUF,
    ],
    compiler_params=pltpu.CompilerParams(use_tc_tiling_on_sc=False),
)
def _gather_kernel(idx_hbm, table_hbm, out_hbm, idx_v, rows_v, idx_sems,
                   gat_sems, out_sems):
    wid = lax.axis_index("s") * _NC + lax.axis_index("c")
    base = wid * _PER_W

    idx_d = [None] * _NBUF
    gat_d = [None] * _NBUF
    out_d = [None] * _NBUF

    def start_idx(i):
        b = i % _NBUF
        idx_d[b] = pltpu.async_copy(
            idx_hbm.at[pl.ds(base + i * _CHUNK, _CHUNK)], idx_v.at[b],
            idx_sems[b])

    def start_gather(i):
        b = i % _NBUF
        gat_d[b] = pltpu.async_copy(
            table_hbm.at[idx_v.at[b]], rows_v.at[b], gat_sems[b])

    def start_out(i):
        b = i % _NBUF
        out_d[b] = pltpu.async_copy(
            rows_v.at[b], out_hbm.at[pl.ds(base + i * _CHUNK, _CHUNK)],
            out_sems[b])

    for i in range(min(_NBUF, _N_CHUNKS)):
        start_idx(i)

    _DEPTH = _NBUF - 1
    for i in range(_N_CHUNKS + _DEPTH):
        if i < _N_CHUNKS:
            b = i % _NBUF
            if out_d[b] is not None:
                out_d[b].wait()
            idx_d[b].wait()
            start_gather(i)
        j = i - _DEPTH
        if 0 <= j:
            bj = j % _NBUF
            gat_d[bj].wait()
            start_out(j)
            if j + _NBUF < _N_CHUNKS:
                start_idx(j + _NBUF)

    for b in range(_NBUF):
        if out_d[b] is not None:
            out_d[b].wait()


def kernel(inputs, W):
    idx = inputs.reshape(-1).astype(jnp.int32)
    out = _gather_kernel(idx, W)
    return out.reshape(inputs.shape + (DIM,))


# speedup vs baseline: 1.7078x; 1.5343x over previous
import functools

import jax
import jax.numpy as jnp
from jax import lax
from jax.experimental import pallas as pl
from jax.experimental.pallas import tpu as pltpu
from jax.experimental.pallas import tpu_sc as plsc

NUM_ROWS = 1_000_000
DIM = 32
NB = 16384
NS = 50

_NW = 32
_TC_PER_W = 4

_mesh = plsc.VectorSubcoreMesh(core_axis_name="c", subcore_axis_name="s")


@functools.partial(
    pl.kernel,
    mesh=_mesh,
    out_type=jax.ShapeDtypeStruct((NS, DIM // 8, NB // 128, 1024),
                                  jnp.float32),
    scratch_types=[
        pltpu.VMEM((2, NS, 128), jnp.int32),
        pltpu.VMEM((2, 128, DIM), jnp.float32),
        pltpu.VMEM((2, 4096), jnp.float32),
        [pltpu.SemaphoreType.DMA] * 2,
        [pltpu.SemaphoreType.DMA] * 2,
        [pltpu.SemaphoreType.DMA] * 2,
    ],
    compiler_params=pltpu.CompilerParams(use_tc_tiling_on_sc=False,
                                         needs_layout_passes=False),
)
def _gather_kernel(idxT_hbm, table_hbm, out_hbm, idx_v, rows_v, panel_v,
                   si, sg, so):
    wid = lax.axis_index("s") * 2 + lax.axis_index("c")

    iota = lax.broadcasted_iota(jnp.int32, (16,), 0)

    def idx_dma(tcp, tc_abs):
        return pltpu.async_copy(
            idxT_hbm.at[:, pl.ds(128 * tc_abs, 128)], idx_v.at[tcp], si[tcp])

    def wait_idx(tcp):
        pltpu.make_async_copy(
            idxT_hbm.at[:, pl.ds(0, 128)], idx_v.at[tcp], si[tcp]).wait()

    def start_gather(sp, tcp, s):
        return pltpu.async_copy(
            table_hbm.at[idx_v.at[tcp, s]], rows_v.at[sp], sg[sp])

    def wait_out(sp):
        for tr in range(4):
            pltpu.make_async_copy(
                panel_v.at[sp, pl.ds(1024 * tr, 1024)],
                out_hbm.at[0, 0, 0], so[sp]).wait()

    def start_out(sp, s, tc_abs):
        for tr in range(4):
            pltpu.async_copy(
                panel_v.at[sp, pl.ds(1024 * tr, 1024)],
                out_hbm.at[s, tr, tc_abs], so[sp])

    def transpose_rows(sp):
        def c_body(bl, carry):
            blsplat = jnp.broadcast_to(bl, (16,)).astype(jnp.int32)
            for h in range(2):
                val = rows_v[sp, bl, pl.ds(16 * h, 16)]
                pos = (16 * h + iota) * 128 + blsplat
                plsc.store_scatter(panel_v.at[sp], [pos], val)
            return carry
        lax.fori_loop(0, 128, c_body, 0)

    first_tc = wid * _TC_PER_W
    idx_dma(0, first_tc)
    for tc in range(_TC_PER_W):
        tcp = tc & 1
        tc_abs = first_tc + tc
        wait_idx(tcp)
        if tc + 1 < _TC_PER_W:
            idx_dma(1 - tcp, tc_abs + 1)

        def pair_body(k, carry, _tcp=tcp, _tc_abs=tc_abs, _first=(tc == 0)):
            s0 = 2 * k
            g0 = start_gather(0, _tcp, s0)
            g1 = start_gather(1, _tcp, s0 + 1)
            if _first:
                @pl.when(k > 0)
                def _():
                    wait_out(0)
            else:
                wait_out(0)
            g0.wait()
            transpose_rows(0)
            start_out(0, s0, _tc_abs)
            if _first:
                @pl.when(k > 0)
                def _():
                    wait_out(1)
            else:
                wait_out(1)
            g1.wait()
            transpose_rows(1)
            start_out(1, s0 + 1, _tc_abs)
            return carry

        lax.fori_loop(0, NS // 2, pair_body, 0)

    wait_out(0)
    wait_out(1)


def kernel(inputs, W):
    idxT = inputs.T.astype(jnp.int32)
    out4 = _gather_kernel(idxT, W).reshape(NS, DIM // 8, NB // 128, 8, 128)
    return out4.transpose(2, 4, 0, 1, 3).reshape(NB, NS, DIM)


# speedup vs baseline: 1.8001x; 1.0541x over previous
import functools

import jax
import jax.numpy as jnp
from jax import lax
from jax.experimental import pallas as pl
from jax.experimental.pallas import tpu as pltpu
from jax.experimental.pallas import tpu_sc as plsc

NUM_ROWS = 1_000_000
DIM = 32
NB = 16384
NS = 50

_TC_PER_W = 4
_SC_PER_CHUNK = 10
_CHUNKS = NS // _SC_PER_CHUNK
_CROWS = 128 * _SC_PER_CHUNK

_mesh = plsc.VectorSubcoreMesh(core_axis_name="c", subcore_axis_name="s")


@functools.partial(
    pl.kernel,
    mesh=_mesh,
    out_type=jax.ShapeDtypeStruct((NS, DIM // 8, NB // 128, 1024),
                                  jnp.float32),
    scratch_types=[
        pltpu.VMEM((6400,), jnp.int32),
        pltpu.VMEM((6400,), jnp.int32),
        pltpu.VMEM((2, _CROWS, DIM), jnp.float32),
        pltpu.VMEM((2, 4096), jnp.float32),
        pltpu.SemaphoreType.DMA,
        [pltpu.SemaphoreType.DMA] * 2,
        [pltpu.SemaphoreType.DMA] * 2,
    ],
    compiler_params=pltpu.CompilerParams(use_tc_tiling_on_sc=False,
                                         needs_layout_passes=False),
)
def _gather_kernel(idx_hbm, table_hbm, out_hbm, raw_v, sm_v, rows_v,
                   panel_v, si, sg, so):
    wid = lax.axis_index("s") * 2 + lax.axis_index("c")

    iota = lax.broadcasted_iota(jnp.int32, (16,), 0)

    def idx_dma(tc_abs):
        return pltpu.async_copy(
            idx_hbm.at[pl.ds(tc_abs * 6400, 6400)], raw_v, si)

    def regroup():
        def body(s, carry):
            splat = jnp.broadcast_to(s, (16,)).astype(jnp.int32)
            for m in range(8):
                pos = (iota + 16 * m) * NS + splat
                sm_v[pl.ds(s * 128 + 16 * m, 16)] = plsc.load_gather(
                    raw_v, [pos])
            return carry
        lax.fori_loop(0, NS, body, 0)

    def start_gather(c):
        return pltpu.async_copy(
            table_hbm.at[sm_v.at[pl.ds(c * _CROWS, _CROWS)]],
            rows_v.at[c & 1], sg[c & 1])

    def wait_out(sp):
        for tr in range(4):
            pltpu.make_async_copy(
                panel_v.at[sp, pl.ds(1024 * tr, 1024)],
                out_hbm.at[0, 0, 0], so[sp]).wait()

    def start_out(sp, s, tc_abs):
        for tr in range(4):
            pltpu.async_copy(
                panel_v.at[sp, pl.ds(1024 * tr, 1024)],
                out_hbm.at[s, tr, tc_abs], so[sp])

    def transpose_one(cp, q, sp):
        def body(j, carry):
            for u in range(8):
                bl = j * 8 + u
                blsplat = jnp.broadcast_to(bl, (16,)).astype(jnp.int32)
                for h in range(2):
                    val = rows_v[cp, q * 128 + bl, pl.ds(16 * h, 16)]
                    pos = (16 * h + iota) * 128 + blsplat
                    plsc.store_scatter(panel_v.at[sp], [pos], val)
            return carry
        lax.fori_loop(0, 16, body, 0)

    first_tc = wid * _TC_PER_W
    idx_dma(first_tc).wait()
    for tc in range(_TC_PER_W):
        tc_abs = first_tc + tc
        regroup()
        if tc + 1 < _TC_PER_W:
            d = idx_dma(tc_abs + 1)
        g = [None] * _CHUNKS
        g[0] = start_gather(0)
        for c in range(_CHUNKS):
            g[c].wait()
            if c + 1 < _CHUNKS:
                g[c + 1] = start_gather(c + 1)

            def q_body(jq, carry, _c=c, _tc=tc, _tc_abs=tc_abs):
                for sp in range(2):
                    q = 2 * jq + sp
                    s = _c * _SC_PER_CHUNK + q
                    if _tc == 0 and _c == 0:
                        @pl.when(jq > 0)
                        def _():
                            wait_out(sp)
                    else:
                        wait_out(sp)
                    transpose_one(_c & 1, q, sp)
                    start_out(sp, s, _tc_abs)
                return carry

            lax.fori_loop(0, _SC_PER_CHUNK // 2, q_body, 0)
        if tc + 1 < _TC_PER_W:
            d.wait()

    wait_out(0)
    wait_out(1)


def kernel(inputs, W):
    idx = inputs.reshape(-1).astype(jnp.int32)
    out4 = _gather_kernel(idx, W).reshape(NS, DIM // 8, NB // 128, 8, 128)
    return out4.transpose(2, 4, 0, 1, 3).reshape(NB, NS, DIM)


# speedup vs baseline: 2.1089x; 1.1715x over previous
import functools

import jax
import jax.numpy as jnp
from jax import lax
from jax.experimental import pallas as pl
from jax.experimental.pallas import tpu as pltpu
from jax.experimental.pallas import tpu_sc as plsc

NUM_ROWS = 1_000_000
DIM = 32
NB = 16384
NS = 50

_TC_PER_W = 4
_SC_PER_CHUNK = 10
_CHUNKS = NS // _SC_PER_CHUNK
_CROWS = 128 * _SC_PER_CHUNK

_mesh = plsc.VectorSubcoreMesh(core_axis_name="c", subcore_axis_name="s")


@functools.partial(
    pl.kernel,
    mesh=_mesh,
    out_type=jax.ShapeDtypeStruct((NS, DIM // 8, NB // 128, 1024),
                                  jnp.float32),
    scratch_types=[
        pltpu.VMEM((6400,), jnp.int32),
        pltpu.VMEM((6400,), jnp.int32),
        pltpu.VMEM((2, _CROWS, DIM), jnp.float32),
        pltpu.VMEM((2, 4096), jnp.float32),
        pltpu.SemaphoreType.DMA,
        [pltpu.SemaphoreType.DMA] * 2,
        [pltpu.SemaphoreType.DMA] * 2,
    ],
    compiler_params=pltpu.CompilerParams(use_tc_tiling_on_sc=False,
                                         needs_layout_passes=False),
)
def _gather_kernel(idx_hbm, table_hbm, out_hbm, raw_v, sm_v, rows_v,
                   panel_v, si, sg, so):
    wid = lax.axis_index("s") * 2 + lax.axis_index("c")

    iota = lax.broadcasted_iota(jnp.int32, (16,), 0)

    def idx_dma(tc_abs):
        return pltpu.async_copy(
            idx_hbm.at[pl.ds(tc_abs * 6400, 6400)], raw_v, si)

    def regroup():
        @plsc.parallel_loop(0, NS, 1, unroll=2)
        def body(s):
            splat = jnp.broadcast_to(s, (16,)).astype(jnp.int32)
            for m in range(8):
                pos = (iota + 16 * m) * NS + splat
                sm_v[pl.ds(s * 128 + 16 * m, 16)] = plsc.load_gather(
                    raw_v, [pos])

    def start_gather(c):
        return pltpu.async_copy(
            table_hbm.at[sm_v.at[pl.ds(c * _CROWS, _CROWS)]],
            rows_v.at[c & 1], sg[c & 1])

    def wait_out(sp):
        for tr in range(4):
            pltpu.make_async_copy(
                panel_v.at[sp, pl.ds(1024 * tr, 1024)],
                out_hbm.at[0, 0, 0], so[sp]).wait()

    def start_out(sp, s, tc_abs):
        for tr in range(4):
            pltpu.async_copy(
                panel_v.at[sp, pl.ds(1024 * tr, 1024)],
                out_hbm.at[s, tr, tc_abs], so[sp])

    def transpose_one(cp, q, sp):
        @plsc.parallel_loop(0, 128, 1, unroll=8)
        def body(bl):
            blsplat = jnp.broadcast_to(bl, (16,)).astype(jnp.int32)
            for h in range(2):
                val = rows_v[cp, q * 128 + bl, pl.ds(16 * h, 16)]
                pos = (16 * h + iota) * 128 + blsplat
                plsc.store_scatter(panel_v.at[sp], [pos], val)

    first_tc = wid * _TC_PER_W
    idx_dma(first_tc).wait()
    for tc in range(_TC_PER_W):
        tc_abs = first_tc + tc
        regroup()
        if tc + 1 < _TC_PER_W:
            d = idx_dma(tc_abs + 1)
        g = [None] * _CHUNKS
        g[0] = start_gather(0)
        for c in range(_CHUNKS):
            g[c].wait()
            if c + 1 < _CHUNKS:
                g[c + 1] = start_gather(c + 1)

            def q_body(jq, carry, _c=c, _tc=tc, _tc_abs=tc_abs):
                for sp in range(2):
                    q = 2 * jq + sp
                    s = _c * _SC_PER_CHUNK + q
                    if _tc == 0 and _c == 0:
                        @pl.when(jq > 0)
                        def _():
                            wait_out(sp)
                    else:
                        wait_out(sp)
                    transpose_one(_c & 1, q, sp)
                    start_out(sp, s, _tc_abs)
                return carry

            lax.fori_loop(0, _SC_PER_CHUNK // 2, q_body, 0)
        if tc + 1 < _TC_PER_W:
            d.wait()

    wait_out(0)
    wait_out(1)


def kernel(inputs, W):
    idx = inputs.reshape(-1).astype(jnp.int32)
    out4 = _gather_kernel(idx, W).reshape(NS, DIM // 8, NB // 128, 8, 128)
    return out4.transpose(2, 4, 0, 1, 3).reshape(NB, NS, DIM)
